# baseline (speedup 1.0000x reference)
import functools

import jax
import jax.numpy as jnp
from jax import lax
from jax.experimental import pallas as pl
from jax.experimental.pallas import tpu as pltpu
from jax.experimental.pallas import tpu_sc as plsc

N_VOCAB = 1000000
N_EMBED = 64
BATCH = 16384
_HALF = N_VOCAB // 2

_info = plsc.get_sparse_core_info()
_NC, _NS, _L = _info.num_cores, _info.num_subcores, _info.num_lanes
_NW = _NS
_BPW = BATCH // _NW
_CHUNK = 128
_NCHUNK = _BPW // _CHUNK

_mesh = plsc.VectorSubcoreMesh(
    core_axis_name="c", subcore_axis_name="s", num_cores=1
)


def _make_half_kernel(lo):
    @functools.partial(
        pl.kernel,
        mesh=_mesh,
        out_type=jax.ShapeDtypeStruct((BATCH, N_EMBED), jnp.float32),
        scratch_types=[
            pltpu.VMEM((_BPW,), jnp.int32),
            pltpu.VMEM((2, _CHUNK, N_EMBED), jnp.float32),
            [pltpu.SemaphoreType.DMA] * 2,
            pltpu.SemaphoreType.DMA,
        ],
        compiler_params=pltpu.CompilerParams(use_tc_tiling_on_sc=False),
        name=f"half_gather_{lo}",
    )
    def _half_kernel(idx_hbm, half_hbm, out_hbm, idx_v, rows_v, gsems, osem):
        wid = lax.axis_index("s")
        base = wid * _BPW
        pltpu.sync_copy(idx_hbm.at[pl.ds(base, _BPW)], idx_v)
        for t in range(_BPW // _L):
            v = idx_v[pl.ds(t * _L, _L)]
            v = lax.clamp(jnp.int32(0), v - lo, jnp.int32(_HALF - 1))
            idx_v[pl.ds(t * _L, _L)] = v

        def fire(j, buf):
            return pltpu.async_copy(
                half_hbm.at[idx_v.at[pl.ds(j * _CHUNK, _CHUNK)]],
                rows_v.at[buf], gsems[buf],
            )

        pending = fire(0, 0)
        out_pending = None
        for j in range(_NCHUNK):
            if out_pending is not None:
                out_pending.wait()
            nxt = None
            if j + 1 < _NCHUNK:
                nxt = fire(j + 1, (j + 1) % 2)
            pending.wait()
            out_pending = pltpu.async_copy(
                rows_v.at[j % 2],
                out_hbm.at[pl.ds(base + j * _CHUNK, _CHUNK)],
                osem,
            )
            pending = nxt
        out_pending.wait()

    return _half_kernel


_gather_lo = _make_half_kernel(0)
_gather_hi = _make_half_kernel(_HALF)


def kernel(input_words, in_embed):
    idx = input_words.astype(jnp.int32)
    lo_rows = _gather_lo(idx, in_embed[:_HALF])
    hi_rows = _gather_hi(idx, in_embed[_HALF:])
    return jnp.where((idx < _HALF)[:, None], lo_rows, hi_rows)


# speedup vs baseline: 3.3319x; 3.3319x over previous
import functools

import jax
import jax.numpy as jnp
from jax import lax
from jax.experimental import pallas as pl
from jax.experimental.pallas import tpu as pltpu
from jax.experimental.pallas import tpu_sc as plsc

N_VOCAB = 1000000
N_EMBED = 64
BATCH = 16384

_info = plsc.get_sparse_core_info()
_NC, _NS, _L = _info.num_cores, _info.num_subcores, _info.num_lanes
_NW = _NC * _NS
_BPW = BATCH // _NW
_CHUNK = 64
_NCHUNK = _BPW // _CHUNK

_mesh = plsc.VectorSubcoreMesh(core_axis_name="c", subcore_axis_name="s")


@functools.partial(
    pl.kernel,
    mesh=_mesh,
    out_type=jax.ShapeDtypeStruct((BATCH, N_EMBED), jnp.float32),
    scratch_types=[
        pltpu.VMEM((_BPW,), jnp.int32),
        pltpu.VMEM((2, _CHUNK, N_EMBED), jnp.float32),
        [pltpu.SemaphoreType.DMA] * 4,
        pltpu.SemaphoreType.DMA,
    ],
)
def _gather_kernel(idx_hbm, tbl_hbm, out_hbm, idx_v, rows_v, sems, osem):
    wid = lax.axis_index("s") * _NC + lax.axis_index("c")
    base = wid * _BPW
    pltpu.sync_copy(idx_hbm.at[pl.ds(base, _BPW)], idx_v)

    def fire(j, buf):
        copies = []
        for g in range(_CHUNK // _L):
            vec = idx_v[pl.ds(j * _CHUNK + g * _L, _L)]
            for k in range(_L):
                i = g * _L + k
                copies.append(
                    pltpu.async_copy(
                        tbl_hbm.at[pl.ds(vec[k], 1)],
                        rows_v.at[buf].at[pl.ds(i, 1)],
                        sems[i % 4],
                    )
                )
        return copies

    pending = fire(0, 0)
    out_pending = None
    for j in range(_NCHUNK):
        if out_pending is not None:
            out_pending.wait()
        nxt = None
        if j + 1 < _NCHUNK:
            nxt = fire(j + 1, (j + 1) % 2)
        for c in pending:
            c.wait()
        out_pending = pltpu.async_copy(
            rows_v.at[j % 2],
            out_hbm.at[pl.ds(base + j * _CHUNK, _CHUNK)],
            osem,
        )
        pending = nxt
    out_pending.wait()


def kernel(input_words, in_embed):
    idx = input_words.astype(jnp.int32)
    return _gather_kernel(idx, in_embed)
